# Initial kernel scaffold; baseline (speedup 1.0000x reference)
import jax
import jax.numpy as jnp
from jax.experimental import pallas as pl


def kernel(input_ids, word_emb, pos_emb, type_emb, lang_emb, ln_w, ln_b):
    raise NotImplementedError("write your pallas kernel here")


# speedup vs baseline: 2.0524x; 2.0524x over previous
import functools

import jax
import jax.numpy as jnp
from jax import lax
from jax.experimental import pallas as pl
from jax.experimental.pallas import tpu as pltpu
from jax.experimental.pallas import tpu_sc as plsc

_EPS = 1e-12
_D = 128
_LANES = 16
_NVREG = _D // _LANES


def _build_sc_kernel(B, seq_len):
    NC, NS = 2, 16
    NW = NC * NS
    CH = B // NW
    assert B % NW == 0 and seq_len % CH == 0
    chunks_per_seq = seq_len // CH

    mesh = plsc.VectorSubcoreMesh(core_axis_name="c", subcore_axis_name="s")

    @functools.partial(
        pl.kernel,
        mesh=mesh,
        compiler_params=pltpu.CompilerParams(needs_layout_passes=False),
        out_type=(
            jax.ShapeDtypeStruct((B, _D), jnp.float32),
            jax.ShapeDtypeStruct((seq_len, _D), jnp.float32),
        ),
        scratch_types=[
            pltpu.VMEM((2, 128), jnp.int32),
            pltpu.VMEM((CH, _D), jnp.float32),
            pltpu.VMEM((CH, _D), jnp.float32),
            pltpu.VMEM((_D,), jnp.float32),
            pltpu.VMEM((_D,), jnp.float32),
            pltpu.VMEM((_D,), jnp.float32),
            pltpu.VMEM((_D,), jnp.float32),
            pltpu.SemaphoreType.DMA,
            pltpu.SemaphoreType.DMA,
            pltpu.SemaphoreType.DMA,
            pltpu.SemaphoreType.DMA,
        ],
    )
    def sc_embed(ids_hbm, wemb_hbm, pemb_hbm, temb_hbm, lemb_hbm, lnw_hbm,
                 lnb_hbm, out1_hbm, out2_hbm,
                 idx_v, rows_v, pos_v, tv, lv, wv, bv,
                 sem_i, sem_g, sem_p, sem_o):
        wid = lax.axis_index("s") * NC + lax.axis_index("c")
        base = wid * CH
        pos_base = lax.rem(wid, chunks_per_seq) * CH

        cp_i0 = pltpu.async_copy(ids_hbm.at[pl.ds(base, 128)], idx_v.at[0], sem_i)
        cp_i1 = pltpu.async_copy(ids_hbm.at[pl.ds(base + 128, 128)], idx_v.at[1], sem_i)
        cp_p = pltpu.async_copy(pemb_hbm.at[pl.ds(pos_base, CH)], pos_v, sem_p)
        cp_i0.wait()
        cp_i1.wait()
        cp_g0 = pltpu.async_copy(wemb_hbm.at[idx_v.at[0]],
                                 rows_v.at[pl.ds(0, 128)], sem_g)
        cp_g1 = pltpu.async_copy(wemb_hbm.at[idx_v.at[1]],
                                 rows_v.at[pl.ds(128, 128)], sem_g)

        pltpu.sync_copy(temb_hbm.at[0], tv)
        pltpu.sync_copy(lemb_hbm.at[1], lv)
        pltpu.sync_copy(lnw_hbm, wv)
        pltpu.sync_copy(lnb_hbm, bv)

        cp_p.wait()
        n_dup = NW // chunks_per_seq
        sub = wid // chunks_per_seq
        sub_rows = CH // n_dup
        sub_off = sub * sub_rows
        cp_o2 = pltpu.async_copy(
            pos_v.at[pl.ds(sub_off, sub_rows)],
            out2_hbm.at[pl.ds(pos_base + sub_off, sub_rows)], sem_o)

        c_reg = [tv[pl.ds(_LANES * j, _LANES)] + lv[pl.ds(_LANES * j, _LANES)]
                 for j in range(_NVREG)]
        w_reg = [wv[pl.ds(_LANES * j, _LANES)] for j in range(_NVREG)]
        b_reg = [bv[pl.ds(_LANES * j, _LANES)] for j in range(_NVREG)]

        cp_g0.wait()
        cp_g1.wait()

        def row_body(r, carry):
            xs = []
            s = jnp.zeros((_LANES,), jnp.float32)
            ss = jnp.zeros((_LANES,), jnp.float32)
            for j in range(_NVREG):
                w = rows_v[r, pl.ds(_LANES * j, _LANES)]
                p = pos_v[r, pl.ds(_LANES * j, _LANES)]
                x = w + p + c_reg[j]
                xs.append(x)
                s = s + x
                ss = ss + x * x
            mean = jnp.sum(s) * (1.0 / _D)
            var = jnp.sum(ss) * (1.0 / _D) - mean * mean
            var = jnp.maximum(var, 0.0) + _EPS
            vv = jnp.zeros((_LANES,), jnp.float32) + var
            yi = jnp.int32(0x5F3759DF) - lax.shift_right_arithmetic(
                plsc.bitcast(vv, jnp.int32), 1)
            y = plsc.bitcast(yi, jnp.float32)
            y = y * (1.5 - 0.5 * vv * y * y)
            y = y * (1.5 - 0.5 * vv * y * y)
            y = y * (1.5 - 0.5 * vv * y * y)
            for j in range(_NVREG):
                rows_v[r, pl.ds(_LANES * j, _LANES)] = (
                    (xs[j] - mean) * y * w_reg[j] + b_reg[j])
            return carry

        lax.fori_loop(0, CH, row_body, 0)

        pltpu.sync_copy(rows_v, out1_hbm.at[pl.ds(base, CH)])
        cp_o2.wait()

    return sc_embed


def kernel(input_ids, word_emb, pos_emb, type_emb, lang_emb, ln_w, ln_b):
    bsz, seq_len = input_ids.shape
    B = bsz * seq_len
    ids = input_ids.reshape(-1).astype(jnp.int32)
    sc_embed = _build_sc_kernel(B, seq_len)
    out1, out2 = sc_embed(ids, word_emb, pos_emb, type_emb, lang_emb,
                          ln_w, ln_b)
    return (out1.reshape(bsz, seq_len, _D), out2.reshape(1, seq_len, _D))
